# Initial kernel scaffold; baseline (speedup 1.0000x reference)
import functools

import jax
import jax.numpy as jnp
from jax import lax
from jax.experimental import pallas as pl
from jax.experimental.pallas import tpu as pltpu
from jax.experimental.pallas import tpu_sc as plsc

B = 16384
L = 50
D = 64
N = B * L

NC = 2
NS = 16
NW = NC * NS

IB = 128
K = 8
CH = IB * K
PER_W = N // NW
N_CHUNKS = PER_W // CH


def _gather_body(idx_hbm, table_hbm, out_hbm, idx_v, rows_v, sem):
    wid = lax.axis_index("s") * NC + lax.axis_index("c")
    base = wid * PER_W

    @pl.loop(0, N_CHUNKS)
    def _chunk(i):
        off = base + i * CH
        pltpu.sync_copy(idx_hbm.at[pl.ds(off // IB, K)], idx_v)
        copies = [
            pltpu.async_copy(
                table_hbm.at[idx_v.at[j]],
                rows_v.at[pl.ds(j * IB, IB)],
                sem,
            )
            for j in range(K)
        ]
        for c in copies:
            c.wait()
        pltpu.sync_copy(rows_v, out_hbm.at[pl.ds(off, CH)])


@jax.jit
def kernel(x, W):
    idx2d = x.astype(jnp.int32).reshape(N // IB, IB)
    mesh = plsc.VectorSubcoreMesh(core_axis_name="c", subcore_axis_name="s")
    out = pl.kernel(
        _gather_body,
        out_type=jax.ShapeDtypeStruct((N, D), jnp.float32),
        mesh=mesh,
        scratch_types=[
            pltpu.VMEM((K, IB), jnp.int32),
            pltpu.VMEM((CH, D), jnp.float32),
            pltpu.SemaphoreType.DMA,
        ],
    )(idx2d, W)
    return out.reshape(B, L, D)


# speedup vs baseline: 1.8446x; 1.8446x over previous
import functools

import jax
import jax.numpy as jnp
from jax import lax
from jax.experimental import pallas as pl
from jax.experimental.pallas import tpu as pltpu
from jax.experimental.pallas import tpu_sc as plsc

B = 16384
L = 50
D = 64
N = B * L

NC = 2
NS = 16
NW = NC * NS

IB = 128
K = 8
CH = IB * K
PER_W = N // NW
N_CHUNKS = PER_W // CH


def _gather_body(idx_hbm, table_hbm, out_hbm, idx_v, rows_v, sem):
    wid = lax.axis_index("s") * NC + lax.axis_index("c")
    base = wid * PER_W

    @pl.loop(0, N_CHUNKS)
    def _chunk(i):
        off = base + i * CH
        row0 = pl.multiple_of(off // IB, 8)
        pltpu.sync_copy(idx_hbm.at[pl.ds(row0, K)], idx_v)
        copies = [
            pltpu.async_copy(
                table_hbm.at[idx_v.at[j]],
                rows_v.at[pl.ds(j * IB, IB)],
                sem,
            )
            for j in range(K)
        ]
        for c in copies:
            c.wait()
        pltpu.sync_copy(rows_v, out_hbm.at[pl.ds(off, CH)])


@jax.jit
def kernel(x, W):
    idx2d = x.astype(jnp.int32).reshape(N // IB, IB)
    mesh = plsc.VectorSubcoreMesh(core_axis_name="c", subcore_axis_name="s")
    out = pl.kernel(
        _gather_body,
        out_type=jax.ShapeDtypeStruct((N, D), jnp.float32),
        mesh=mesh,
        compiler_params=pltpu.CompilerParams(use_tc_tiling_on_sc=False),
        scratch_types=[
            pltpu.VMEM((K, IB), jnp.int32),
            pltpu.VMEM((CH, D), jnp.float32),
            pltpu.SemaphoreType.DMA,
        ],
    )(idx2d, W)
    return out.reshape(B, L, D)


# speedup vs baseline: 1.8737x; 1.0158x over previous
import jax
import jax.numpy as jnp
from jax import lax
from jax.experimental import pallas as pl
from jax.experimental.pallas import tpu as pltpu
from jax.experimental.pallas import tpu_sc as plsc

B = 16384
L = 50
D = 64
N = B * L

NC = 2
NS = 16
NW = NC * NS

IB = 128
K = 5
CH = IB * K
PER_W = N // NW
IDX_ROWS = PER_W // IB
N_CHUNKS = PER_W // CH
CHUNK_BYTES = CH * D * 4


def _gather_body(idx_hbm, table_hbm, out_hbm, idx_v, rows_v, sem_g0, sem_g1,
                 sem_o):
    wid = lax.axis_index("s") * NC + lax.axis_index("c")
    base = wid * PER_W
    sem_g = (sem_g0, sem_g1)

    row0 = pl.multiple_of(wid * IDX_ROWS, 8)
    pltpu.sync_copy(idx_hbm.at[pl.ds(row0, IDX_ROWS)], idx_v)

    def fire_gathers(chunk, b):
        for j in range(K):
            pltpu.async_copy(
                table_hbm.at[idx_v.at[chunk * K + j]],
                rows_v.at[b].at[pl.ds(j * IB, IB)],
                sem_g[b],
            )

    def wait_gathers(b):
        pltpu.make_async_copy(
            out_hbm.at[pl.ds(0, CH)], rows_v.at[b], sem_g[b]
        ).wait()

    def wait_out():
        pltpu.make_async_copy(
            rows_v.at[0], out_hbm.at[pl.ds(0, CH)], sem_o
        ).wait()

    fire_gathers(0, 0)

    @pl.loop(0, N_CHUNKS, step=2)
    def _pair(i):
        for b in range(2):
            chunk = i + b
            @pl.when(chunk + 1 < N_CHUNKS)
            def _fire_next():
                @pl.when(chunk >= 1)
                def _w():
                    wait_out()
                fire_gathers(chunk + 1, 1 - b)

            wait_gathers(b)
            pltpu.async_copy(
                rows_v.at[b], out_hbm.at[pl.ds(base + chunk * CH, CH)], sem_o
            )

    wait_out()
    wait_out()


@jax.jit
def kernel(x, W):
    idx2d = x.astype(jnp.int32).reshape(N // IB, IB)
    mesh = plsc.VectorSubcoreMesh(core_axis_name="c", subcore_axis_name="s")
    out = pl.kernel(
        _gather_body,
        out_type=jax.ShapeDtypeStruct((N, D), jnp.float32),
        mesh=mesh,
        compiler_params=pltpu.CompilerParams(use_tc_tiling_on_sc=False),
        scratch_types=[
            pltpu.VMEM((IDX_ROWS, IB), jnp.int32),
            pltpu.VMEM((2, CH, D), jnp.float32),
            pltpu.SemaphoreType.DMA,
            pltpu.SemaphoreType.DMA,
            pltpu.SemaphoreType.DMA,
        ],
    )(idx2d, W)
    return out.reshape(B, L, D)
